# baseline (speedup 1.0000x reference)
import functools

import jax
import jax.numpy as jnp
from jax import lax
from jax.experimental import pallas as pl
from jax.experimental.pallas import tpu as pltpu
from jax.experimental.pallas import tpu_sc as plsc

_B, _C, _H, _W = 2, 96, 224, 112
_HO, _WO = 224, 224
_PLANE = _HO * _WO
_HW = _H * _W
_NC, _NS, _L = 2, 16, 16
_NW = _NC * _NS
_P = _B * _C
_PPW = _P // _NW
_UZ = 16
_US = 16
_CB = 8

_mesh = plsc.VectorSubcoreMesh(core_axis_name="c", subcore_axis_name="s")


@functools.partial(
    pl.kernel,
    mesh=_mesh,
    out_type=jax.ShapeDtypeStruct((_B, 2 * _C, _HO, _WO), jnp.float32),
    scratch_types=[
        pltpu.VMEM((_H, _W), jnp.float32),
        pltpu.VMEM((_H, _W), jnp.int32),
        pltpu.VMEM((_HO, _WO), jnp.float32),
        pltpu.SemaphoreType.DMA,
        pltpu.SemaphoreType.DMA,
    ],
    compiler_params=pltpu.CompilerParams(needs_layout_passes=False),
)
def _sc_unpool(x_hbm, idx_hbm, out_hbm, x_v, idx_v, out_v, sem_in, sem_out):
    wid = lax.axis_index("s") * _NC + lax.axis_index("c")

    def zero_body(h, carry):
        for u in range(_WO // _L):
            out_v[h, pl.ds(u * _L, _L)] = jnp.zeros((_L,), jnp.float32)
        return carry

    def scatter_body(h, carry):
        ivs = [idx_v[h, pl.ds(u * _L, _L)] for u in range(_W // _L)]
        xvs = [x_v[h, pl.ds(u * _L, _L)] for u in range(_W // _L)]
        for u in range(_W // _L):
            ih = ivs[u] // _WO
            iw = ivs[u] - ih * _WO
            plsc.store_scatter(out_v, [ih, iw], xvs[u])
        return carry

    def issue_loads(j):
        p = wid * _PPW + j
        b = p // _C
        c = p - b * _C
        hx = pltpu.async_copy(x_hbm.at[b, c], x_v, sem_in)
        hi = pltpu.async_copy(idx_hbm.at[b, c], idx_v, sem_in)
        return hx, hi

    out_handle = None
    loads = issue_loads(0)
    for j in range(_PPW):
        p = wid * _PPW + j
        b = p // _C
        c = p - b * _C

        if out_handle is not None:
            out_handle.wait()
        lax.fori_loop(0, _HO, zero_body, 0)
        hx, hi = loads
        hx.wait()
        hi.wait()
        lax.fori_loop(0, _H, scatter_body, 0)
        out_handle = pltpu.async_copy(out_v, out_hbm.at[b, c], sem_out)
        if j + 1 < _PPW:
            loads = issue_loads(j + 1)
    out_handle.wait()


def _tc_pre_body(pre_ref, alias_ref, out_ref):
    del alias_ref
    out_ref[...] = pre_ref[...]


_tc_pre = pl.pallas_call(
    _tc_pre_body,
    grid=(_B, _C // _CB),
    in_specs=[
        pl.BlockSpec((1, _CB, _HO, _WO), lambda b, j: (b, j, 0, 0)),
        pl.BlockSpec(memory_space=pl.ANY),
    ],
    out_specs=pl.BlockSpec(
        (1, _CB, _HO, _WO), lambda b, j: (b, _C // _CB + j, 0, 0)),
    out_shape=jax.ShapeDtypeStruct((_B, 2 * _C, _HO, _WO), jnp.float32),
    input_output_aliases={1: 0},
)


def kernel(x, indices, pre_x):
    scattered = _sc_unpool(x, indices.astype(jnp.int32))
    return _tc_pre(pre_x, scattered)


# speedup vs baseline: 1.8170x; 1.8170x over previous
import functools

import jax
import jax.numpy as jnp
from jax import lax
from jax.experimental import pallas as pl
from jax.experimental.pallas import tpu as pltpu
from jax.experimental.pallas import tpu_sc as plsc

_B, _C, _H, _W = 2, 96, 224, 112
_HO, _WO = 224, 224
_PLANE = _HO * _WO
_HW = _H * _W
_NC, _NS, _L = 2, 16, 16
_NW = _NC * _NS
_P = _B * _C
_PPW = _P // _NW
_UZ = 16
_SUBL = _PLANE // 128

_mesh = plsc.VectorSubcoreMesh(core_axis_name="c", subcore_axis_name="s")


@functools.partial(
    pl.kernel,
    mesh=_mesh,
    out_type=jax.ShapeDtypeStruct((_B * 2 * _C * _PLANE,), jnp.float32),
    scratch_types=[
        pltpu.VMEM((_H, _W), jnp.float32),
        pltpu.VMEM((_H, _W), jnp.int32),
        pltpu.VMEM((_PLANE,), jnp.float32),
        pltpu.SemaphoreType.DMA,
        pltpu.SemaphoreType.DMA,
    ],
    compiler_params=pltpu.CompilerParams(needs_layout_passes=False),
)
def _sc_unpool(x_hbm, idx_hbm, out_hbm, x_v, idx_v, out_v, sem_in, sem_out):
    wid = lax.axis_index("s") * _NC + lax.axis_index("c")

    def zero_body(i, carry):
        base = i * (_L * _UZ)
        for u in range(_UZ):
            out_v[pl.ds(base + u * _L, _L)] = jnp.zeros((_L,), jnp.float32)
        return carry

    def scatter_body(h, carry):
        ivs = [idx_v[h, pl.ds(u * _L, _L)] for u in range(_W // _L)]
        xvs = [x_v[h, pl.ds(u * _L, _L)] for u in range(_W // _L)]
        for u in range(_W // _L):
            plsc.store_scatter(out_v, [ivs[u]], xvs[u])
        return carry

    def issue_loads(j):
        p = wid * _PPW + j
        b = p // _C
        c = p - b * _C
        hx = pltpu.async_copy(x_hbm.at[b, c], x_v, sem_in)
        hi = pltpu.async_copy(idx_hbm.at[b, c], idx_v, sem_in)
        return hx, hi

    out_handle = None
    loads = issue_loads(0)
    for j in range(_PPW):
        p = wid * _PPW + j
        b = p // _C
        c = p - b * _C
        row_u = b * (2 * _C) + c

        if out_handle is not None:
            out_handle.wait()
        lax.fori_loop(0, _PLANE // (_L * _UZ), zero_body, 0)
        hx, hi = loads
        hx.wait()
        hi.wait()
        lax.fori_loop(0, _H, scatter_body, 0)
        out_handle = pltpu.async_copy(
            out_v, out_hbm.at[pl.ds(row_u * _PLANE, _PLANE)], sem_out)
        if j + 1 < _PPW:
            loads = issue_loads(j + 1)
    out_handle.wait()


_CB = 16


def _tc_pre_body(pre_ref, alias_ref, out_ref):
    del alias_ref
    out_ref[...] = pre_ref[...]


_tc_pre = pl.pallas_call(
    _tc_pre_body,
    grid=(_B, _C // _CB),
    in_specs=[
        pl.BlockSpec((1, _CB, _HO, _WO), lambda b, j: (b, j, 0, 0)),
        pl.BlockSpec(memory_space=pl.ANY),
    ],
    out_specs=pl.BlockSpec(
        (1, _CB, _HO, _WO), lambda b, j: (b, _C // _CB + j, 0, 0)),
    out_shape=jax.ShapeDtypeStruct((_B, 2 * _C, _HO, _WO), jnp.float32),
    input_output_aliases={1: 0},
)


def kernel(x, indices, pre_x):
    B, C = x.shape[0], x.shape[1]
    scattered = _sc_unpool(x, indices.astype(jnp.int32))
    return _tc_pre(pre_x, scattered.reshape(B, 2 * C, _HO, _WO))
